# Initial kernel scaffold; baseline (speedup 1.0000x reference)
import jax
import jax.numpy as jnp
from jax.experimental import pallas as pl


def kernel(words, postags, word_table, pos_table, gamma, beta):
    raise NotImplementedError("write your pallas kernel here")


# speedup vs baseline: 2.2955x; 2.2955x over previous
import functools

import jax
import jax.numpy as jnp
from jax import lax
from jax.experimental import pallas as pl
from jax.experimental.pallas import tpu as pltpu
from jax.experimental.pallas import tpu_sc as plsc

DIM_WORD = 128
DIM_POS = 64
DIM_TOT = DIM_WORD + DIM_POS
NVREG = DIM_TOT // 16
EPS = 1e-6
T = 128
MAGIC = 0x5F3759DF


def _ln_chunk(wrows, pidx, c, ptab, obuf, gv, bv):
    lanes = jnp.arange(16, dtype=jnp.int32)

    def grp(g, carry):
        pvec = pidx[c, pl.ds(16 * g, 16)] * DIM_POS
        for j in range(16):
            t = 16 * g + j
            vs = []
            acc = jnp.zeros((16,), jnp.float32)
            accq = jnp.zeros((16,), jnp.float32)
            for d in range(DIM_WORD // 16):
                v = wrows[t, pl.ds(16 * d, 16)]
                vs.append(v)
                acc = acc + v
                accq = accq + v * v
            pbase = pvec[j] + lanes
            for d in range(DIM_POS // 16):
                v = plsc.load_gather(ptab, [pbase + 16 * d])
                vs.append(v)
                acc = acc + v
                accq = accq + v * v
            mean = jnp.sum(acc) * (1.0 / DIM_TOT)
            sq = jnp.sum(accq) * (1.0 / DIM_TOT)
            mv = jnp.full((16,), mean, jnp.float32)
            xv = jnp.full((16,), sq - mean * mean + EPS, jnp.float32)
            iv = plsc.bitcast(xv, jnp.int32)
            yv = plsc.bitcast(MAGIC - (iv >> 1), jnp.float32)
            for _ in range(3):
                yv = yv * (1.5 - 0.5 * xv * yv * yv)
            for d in range(NVREG):
                obuf[t, pl.ds(16 * d, 16)] = (vs[d] - mv) * yv * gv[d] + bv[d]
        return carry

    lax.fori_loop(0, T // 16, grp, 0)


def _make_kernel(nw, chunks):
    mesh = plsc.VectorSubcoreMesh(core_axis_name="c", subcore_axis_name="s")
    info = plsc.get_sparse_core_info()
    nc = info.num_cores

    @functools.partial(
        pl.kernel,
        mesh=mesh,
        out_type=jax.ShapeDtypeStruct((nw * chunks * T, DIM_TOT), jnp.float32),
        scratch_types=[
            pltpu.VMEM((chunks, T), jnp.int32),
            pltpu.VMEM((chunks, T), jnp.int32),
            pltpu.VMEM((T, DIM_WORD), jnp.float32),
            pltpu.VMEM((64 * DIM_POS,), jnp.float32),
            pltpu.VMEM((T, DIM_TOT), jnp.float32),
            pltpu.VMEM((DIM_TOT,), jnp.float32),
            pltpu.VMEM((DIM_TOT,), jnp.float32),
            pltpu.SemaphoreType.DMA,
        ],
        compiler_params=pltpu.CompilerParams(needs_layout_passes=False),
    )
    def k(words_hbm, pos_hbm, wtab_hbm, ptab_hbm, gamma_hbm, beta_hbm,
          out_hbm, widx, pidx, wrows, ptab, obuf, gvm, bvm, wsem):
        wid = lax.axis_index("s") * nc + lax.axis_index("c")
        pltpu.sync_copy(words_hbm.at[wid], widx)
        pltpu.sync_copy(pos_hbm.at[wid], pidx)
        pltpu.sync_copy(ptab_hbm, ptab)
        pltpu.sync_copy(gamma_hbm, gvm)
        pltpu.sync_copy(beta_hbm, bvm)
        gv = [gvm[pl.ds(16 * d, 16)] for d in range(NVREG)]
        bv = [bvm[pl.ds(16 * d, 16)] for d in range(NVREG)]

        def chunk(c, carry):
            pltpu.async_copy(wtab_hbm.at[widx.at[c]], wrows, wsem).wait()
            _ln_chunk(wrows, pidx, c, ptab, obuf, gv, bv)
            pltpu.sync_copy(obuf, out_hbm.at[pl.ds((wid * chunks + c) * T, T)])
            return carry

        lax.fori_loop(0, chunks, chunk, 0)

    return k


def kernel(words, postags, word_table, pos_table, gamma, beta):
    B, L = words.shape
    tokens = B * L
    nw = 32
    chunks = tokens // (nw * T)
    widx = words.reshape(nw, chunks, T).astype(jnp.int32)
    pidx = postags.reshape(nw, chunks, T).astype(jnp.int32)
    k = _make_kernel(nw, chunks)
    out = k(widx, pidx, word_table, pos_table.reshape(-1), gamma, beta)
    return out.reshape(B, L, DIM_TOT)


# speedup vs baseline: 2.6634x; 1.1602x over previous
import functools

import jax
import jax.numpy as jnp
from jax import lax
from jax.experimental import pallas as pl
from jax.experimental.pallas import tpu as pltpu
from jax.experimental.pallas import tpu_sc as plsc

DIM_WORD = 128
DIM_POS = 64
DIM_TOT = DIM_WORD + DIM_POS
NVREG = DIM_TOT // 16
EPS = 1e-6
T = 128
MAGIC = 0x5F3759DF


def _ln_chunk(wrows, pidx, c, ptab, obuf, gv, bv):
    lanes = jnp.arange(16, dtype=jnp.int32)

    def grp(g, carry):
        pvec = pidx[c, pl.ds(16 * g, 16)] * DIM_POS
        for j in range(16):
            t = 16 * g + j
            vs = []
            acc = jnp.zeros((16,), jnp.float32)
            accq = jnp.zeros((16,), jnp.float32)
            for d in range(DIM_WORD // 16):
                v = wrows[t, pl.ds(16 * d, 16)]
                vs.append(v)
                acc = acc + v
                accq = accq + v * v
            pbase = pvec[j] + lanes
            for d in range(DIM_POS // 16):
                v = plsc.load_gather(ptab, [pbase + 16 * d])
                vs.append(v)
                acc = acc + v
                accq = accq + v * v
            mean = jnp.sum(acc) * (1.0 / DIM_TOT)
            sq = jnp.sum(accq) * (1.0 / DIM_TOT)
            mv = jnp.full((16,), mean, jnp.float32)
            xv = jnp.full((16,), sq - mean * mean + EPS, jnp.float32)
            iv = plsc.bitcast(xv, jnp.int32)
            yv = plsc.bitcast(MAGIC - (iv >> 1), jnp.float32)
            for _ in range(3):
                yv = yv * (1.5 - 0.5 * xv * yv * yv)
            for d in range(NVREG):
                obuf[t, pl.ds(16 * d, 16)] = (vs[d] - mv) * yv * gv[d] + bv[d]
        return carry

    lax.fori_loop(0, T // 16, grp, 0)


def _make_kernel(nw, chunks):
    mesh = plsc.VectorSubcoreMesh(core_axis_name="c", subcore_axis_name="s")
    info = plsc.get_sparse_core_info()
    nc = info.num_cores

    @functools.partial(
        pl.kernel,
        mesh=mesh,
        out_type=jax.ShapeDtypeStruct((nw * chunks * T, DIM_TOT), jnp.float32),
        scratch_types=[
            pltpu.VMEM((chunks, T), jnp.int32),
            pltpu.VMEM((chunks, T), jnp.int32),
            pltpu.VMEM((T, DIM_WORD), jnp.float32),
            pltpu.VMEM((T, DIM_WORD), jnp.float32),
            pltpu.VMEM((64 * DIM_POS,), jnp.float32),
            pltpu.VMEM((T, DIM_TOT), jnp.float32),
            pltpu.VMEM((T, DIM_TOT), jnp.float32),
            pltpu.VMEM((DIM_TOT,), jnp.float32),
            pltpu.VMEM((DIM_TOT,), jnp.float32),
            pltpu.SemaphoreType.DMA,
            pltpu.SemaphoreType.DMA,
            pltpu.SemaphoreType.DMA,
            pltpu.SemaphoreType.DMA,
        ],
        compiler_params=pltpu.CompilerParams(needs_layout_passes=False),
    )
    def k(words_hbm, pos_hbm, wtab_hbm, ptab_hbm, gamma_hbm, beta_hbm,
          out_hbm, widx, pidx, wrows0, wrows1, ptab, obuf0, obuf1,
          gvm, bvm, wsem0, wsem1, osem0, osem1):
        wid = lax.axis_index("s") * nc + lax.axis_index("c")
        wrows = (wrows0, wrows1)
        obufs = (obuf0, obuf1)
        wsems = (wsem0, wsem1)
        osems = (osem0, osem1)
        pltpu.sync_copy(words_hbm.at[wid], widx)
        pltpu.sync_copy(pos_hbm.at[wid], pidx)
        pltpu.sync_copy(ptab_hbm, ptab)
        pltpu.sync_copy(gamma_hbm, gvm)
        pltpu.sync_copy(beta_hbm, bvm)
        gv = [gvm[pl.ds(16 * d, 16)] for d in range(NVREG)]
        bv = [bvm[pl.ds(16 * d, 16)] for d in range(NVREG)]
        obase = wid * chunks

        def gather(c, b):
            return pltpu.make_async_copy(
                wtab_hbm.at[widx.at[c]], wrows[b], wsems[b])

        def store(c, b):
            return pltpu.make_async_copy(
                obufs[b], out_hbm.at[pl.ds((obase + c) * T, T)], osems[b])

        for b in range(2):
            gather(b, b).start()

        def body(g, carry):
            for b in range(2):
                c = 2 * g + b
                gather(c, b).wait()

                @pl.when(g > 0)
                def _():
                    store(c - 2, b).wait()

                _ln_chunk(wrows[b], pidx, c, ptab, obufs[b], gv, bv)

                @pl.when(c + 2 < chunks)
                def _():
                    gather(c + 2, b).start()

                store(c, b).start()
            return carry

        lax.fori_loop(0, chunks // 2, body, 0)
        for b in range(2):
            store(chunks - 2 + b, b).wait()

    return k


def kernel(words, postags, word_table, pos_table, gamma, beta):
    B, L = words.shape
    tokens = B * L
    nw = 32
    chunks = tokens // (nw * T)
    widx = words.reshape(nw, chunks, T).astype(jnp.int32)
    pidx = postags.reshape(nw, chunks, T).astype(jnp.int32)
    k = _make_kernel(nw, chunks)
    out = k(widx, pidx, word_table, pos_table.reshape(-1), gamma, beta)
    return out.reshape(B, L, DIM_TOT)
